# baseline (speedup 1.0000x reference)
import functools

import jax
import jax.numpy as jnp
from jax import lax
from jax.experimental import pallas as pl
from jax.experimental.pallas import tpu as pltpu
from jax.experimental.pallas import tpu_sc as plsc

B = 16384
EMB = 16
CH = 128


def _make_sc_gather():
    info = plsc.get_sparse_core_info()
    nw = info.num_cores * info.num_subcores
    bpw = B // nw
    nch = bpw // CH
    mesh = plsc.VectorSubcoreMesh(core_axis_name="c", subcore_axis_name="s")

    @functools.partial(
        pl.kernel,
        mesh=mesh,
        out_type=(
            jax.ShapeDtypeStruct((B, EMB), jnp.float32),
            jax.ShapeDtypeStruct((B, EMB), jnp.float32),
        ),
        scratch_types=[
            pltpu.VMEM((nch, CH), jnp.int32),
            pltpu.VMEM((nch, CH), jnp.int32),
            pltpu.VMEM((bpw, EMB), jnp.float32),
            pltpu.VMEM((bpw, EMB), jnp.float32),
            pltpu.SemaphoreType.DMA,
        ],
        compiler_params=pltpu.CompilerParams(use_tc_tiling_on_sc=False),
    )
    def sc_gather(ut_hbm, it_hbm, ui_hbm, ii_hbm, uo_hbm, io_hbm,
                  uix, iix, urows, irows, sem):
        wid = lax.axis_index("s") * info.num_cores + lax.axis_index("c")
        base = wid * bpw
        pltpu.sync_copy(ui_hbm.at[pl.ds(wid * nch, nch)], uix)
        pltpu.sync_copy(ii_hbm.at[pl.ds(wid * nch, nch)], iix)
        copies = []
        for c in range(nch):
            copies.append(pltpu.async_copy(
                ut_hbm.at[uix.at[c]], urows.at[pl.ds(c * CH, CH)], sem))
            copies.append(pltpu.async_copy(
                it_hbm.at[iix.at[c]], irows.at[pl.ds(c * CH, CH)], sem))
        for cp in copies:
            cp.wait()
        pltpu.sync_copy(urows, uo_hbm.at[pl.ds(base, bpw)])
        pltpu.sync_copy(irows, io_hbm.at[pl.ds(base, bpw)])

    return sc_gather


def _tc_mlp_body(u_ref, v_ref, wa_ref, wb_ref, wc_ref, b1_ref, w2_ref, b2_ref,
                 o_ref):
    u = u_ref[...]
    v = v_ref[...]
    e = u * v
    h = (jnp.dot(e, wa_ref[...], preferred_element_type=jnp.float32)
         + jnp.dot(u, wb_ref[...], preferred_element_type=jnp.float32)
         + jnp.dot(v, wc_ref[...], preferred_element_type=jnp.float32)
         + b1_ref[...])
    h = jnp.maximum(h, 0.0)
    y = jnp.dot(h, w2_ref[...], preferred_element_type=jnp.float32) + b2_ref[...]
    o_ref[...] = jax.nn.sigmoid(y)


def kernel(group_inputs, user_inputs, item_inputs, user_table, item_table,
           W1, b1, W2, b2):
    del group_inputs
    uidx = user_inputs.astype(jnp.int32).reshape(B // CH, CH)
    iidx = item_inputs.astype(jnp.int32).reshape(B // CH, CH)
    urows, irows = _make_sc_gather()(user_table, item_table, uidx, iidx)
    y = pl.pallas_call(
        _tc_mlp_body,
        out_shape=jax.ShapeDtypeStruct((B, 1), jnp.float32),
    )(urows, irows, W1[0:EMB], W1[EMB:2 * EMB], W1[2 * EMB:3 * EMB],
      b1.reshape(1, 8), W2, b2.reshape(1, 1))
    return y


# speedup vs baseline: 1.0186x; 1.0186x over previous
import functools

import jax
import jax.numpy as jnp
from jax import lax
from jax.experimental import pallas as pl
from jax.experimental.pallas import tpu as pltpu
from jax.experimental.pallas import tpu_sc as plsc

B = 16384
EMB = 16
PACK = 128 // EMB
CH = 128


def _make_sc_gather(rows_packed):
    info = plsc.get_sparse_core_info()
    nw = info.num_cores * info.num_subcores
    bpw = B // nw
    nch = bpw // CH
    opw = bpw // PACK
    mesh = plsc.VectorSubcoreMesh(core_axis_name="c", subcore_axis_name="s")

    @functools.partial(
        pl.kernel,
        mesh=mesh,
        out_type=(
            jax.ShapeDtypeStruct((B // PACK, 128), jnp.float32),
            jax.ShapeDtypeStruct((B // PACK, 128), jnp.float32),
        ),
        scratch_types=[
            pltpu.VMEM((nch, CH), jnp.int32),
            pltpu.VMEM((nch, CH), jnp.int32),
            pltpu.VMEM((nch, CH), jnp.int32),
            pltpu.VMEM((nch, CH), jnp.int32),
            pltpu.VMEM((2, CH, 128), jnp.float32),
            pltpu.VMEM((2, CH, 128), jnp.float32),
            pltpu.VMEM((opw, 128), jnp.float32),
            pltpu.VMEM((opw, 128), jnp.float32),
            pltpu.SemaphoreType.DMA,
            pltpu.SemaphoreType.DMA,
        ],
        compiler_params=pltpu.CompilerParams(use_tc_tiling_on_sc=True,
                                             needs_layout_passes=False),
    )
    def sc_gather(ut_hbm, it_hbm, ui_hbm, ii_hbm, uo_hbm, io_hbm,
                  uix, iix, urix, irix, ubuf, ibuf, uout, iout, sem0, sem1):
        wid = lax.axis_index("s") * info.num_cores + lax.axis_index("c")
        pltpu.sync_copy(ui_hbm.at[pl.ds(wid * nch, nch)], uix)
        pltpu.sync_copy(ii_hbm.at[pl.ds(wid * nch, nch)], iix)
        for c in range(nch):
            for k in range(CH // 16):
                s = pl.ds(k * 16, 16)
                urix[c, s] = lax.shift_right_logical(uix[c, s], PACK.bit_length() - 1)
                irix[c, s] = lax.shift_right_logical(iix[c, s], PACK.bit_length() - 1)

        sems = (sem0, sem1)

        def fire(c):
            b = c % 2
            return (
                pltpu.async_copy(ut_hbm.at[urix.at[c]], ubuf.at[b], sems[b]),
                pltpu.async_copy(it_hbm.at[irix.at[c]], ibuf.at[b], sems[b]),
            )

        def extract(c, idx_ref, buf, out_ref):
            b = c % 2
            lanes = lax.iota(jnp.int32, 16)

            def group(g, _):
                rows16 = g * 16 + lanes
                sub = idx_ref[c, pl.ds(g * 16, 16)] & (PACK - 1)
                col0 = sub * EMB
                dbase = (c * CH + rows16) * EMB
                for t in range(EMB):
                    vals = plsc.load_gather(buf.at[b], [rows16, col0 + t])
                    df = dbase + t
                    plsc.store_scatter(
                        out_ref,
                        [lax.shift_right_logical(df, 7), df & 127],
                        vals)
                return _

            lax.fori_loop(0, CH // 16, group, None)

        inflight = fire(0)
        for c in range(nch):
            nxt = fire(c + 1) if c + 1 < nch else None
            for cp in inflight:
                cp.wait()
            extract(c, uix, ubuf, uout)
            extract(c, iix, ibuf, iout)
            inflight = nxt

        pltpu.sync_copy(uout, uo_hbm.at[pl.ds(wid * opw, opw)])
        pltpu.sync_copy(iout, io_hbm.at[pl.ds(wid * opw, opw)])

    del rows_packed
    return sc_gather


def _tc_mlp_body(u_ref, v_ref, wa_ref, wb_ref, wc_ref, b1_ref, w2_ref, b2_ref,
                 o_ref):
    u = u_ref[...]
    v = v_ref[...]
    e = u * v
    h = (jnp.dot(e, wa_ref[...], preferred_element_type=jnp.float32)
         + jnp.dot(u, wb_ref[...], preferred_element_type=jnp.float32)
         + jnp.dot(v, wc_ref[...], preferred_element_type=jnp.float32)
         + b1_ref[...])
    h = jnp.maximum(h, 0.0)
    y = jnp.dot(h, w2_ref[...], preferred_element_type=jnp.float32) + b2_ref[...]
    o_ref[...] = jax.nn.sigmoid(y)


def kernel(group_inputs, user_inputs, item_inputs, user_table, item_table,
           W1, b1, W2, b2):
    del group_inputs
    n_users = user_table.shape[0]
    n_items = item_table.shape[0]
    ut_p = user_table.reshape(n_users * EMB // 128, 128)
    it_p = item_table.reshape(n_items * EMB // 128, 128)
    uidx = user_inputs.astype(jnp.int32).reshape(B // CH, CH)
    iidx = item_inputs.astype(jnp.int32).reshape(B // CH, CH)
    uo_p, io_p = _make_sc_gather(ut_p.shape[0])(ut_p, it_p, uidx, iidx)

    eye = jnp.eye(PACK, dtype=jnp.float32)
    wa = jnp.kron(eye, W1[0:EMB])
    wb = jnp.kron(eye, W1[EMB:2 * EMB])
    wc = jnp.kron(eye, W1[2 * EMB:3 * EMB])
    b1t = jnp.tile(b1, PACK).reshape(1, PACK * 8)
    w2p = jnp.kron(eye, W2)
    b2t = jnp.tile(b2, PACK).reshape(1, PACK)

    y = pl.pallas_call(
        _tc_mlp_body,
        out_shape=jax.ShapeDtypeStruct((B // PACK, PACK), jnp.float32),
    )(uo_p, io_p, wa, wb, wc, b1t, w2p, b2t)
    return y.reshape(B, 1)


# speedup vs baseline: 5.9715x; 5.8622x over previous
import functools

import jax
import jax.numpy as jnp
from jax import lax
from jax.experimental import pallas as pl
from jax.experimental.pallas import tpu as pltpu
from jax.experimental.pallas import tpu_sc as plsc

B = 16384
EMB = 16
PACK = 128 // EMB
CHUNK = 16


def _make_sc_gather(n_rows):
    info = plsc.get_sparse_core_info()
    nw = info.num_cores * info.num_subcores
    bpw = B // nw
    nch = bpw // CHUNK
    opw = bpw // PACK
    max_grp = (n_rows - 128) // 128
    mesh = plsc.VectorSubcoreMesh(core_axis_name="c", subcore_axis_name="s")

    @functools.partial(
        pl.kernel,
        mesh=mesh,
        out_type=(
            jax.ShapeDtypeStruct((B // PACK, 128), jnp.float32),
            jax.ShapeDtypeStruct((B // PACK, 128), jnp.float32),
        ),
        scratch_types=[
            pltpu.VMEM((bpw // 128, 128), jnp.int32),
            pltpu.VMEM((bpw // 128, 128), jnp.int32),
            pltpu.VMEM((2, CHUNK, EMB, 128), jnp.float32),
            pltpu.VMEM((opw, 128), jnp.float32),
            pltpu.SemaphoreType.DMA,
            pltpu.SemaphoreType.DMA,
        ],
        compiler_params=pltpu.CompilerParams(use_tc_tiling_on_sc=True,
                                             needs_layout_passes=False),
    )
    def sc_gather(utT_hbm, itT_hbm, ui_hbm, ii_hbm, uo_hbm, io_hbm,
                  uix, iix, gring, stage, sem0, sem1):
        wid = lax.axis_index("s") * info.num_cores + lax.axis_index("c")
        obase = pl.multiple_of(wid * opw, opw)
        lanes = lax.iota(jnp.int32, 16)
        sems = (sem0, sem1)
        nrow = bpw // 128

        def run_table(tbl_hbm, idx_hbm, out_hbm, ixv):
            pltpu.sync_copy(idx_hbm.at[pl.ds(wid * nrow, nrow)], ixv)

            def chunk_idx(c):
                return ixv[(c * CHUNK) // 128,
                           pl.ds((c * CHUNK) % 128, CHUNK)]

            def fire(c, h):
                gv = jnp.minimum(
                    lax.shift_right_logical(chunk_idx(c), 7), max_grp)
                for k in range(CHUNK):
                    col = pl.multiple_of(gv[k] * 128, 128)
                    pltpu.async_copy(tbl_hbm.at[:, pl.ds(col, 128)],
                                     gring.at[h, k], sems[h])

            def drain(h):
                for k in range(CHUNK):
                    pltpu.make_async_copy(tbl_hbm.at[:, pl.ds(0, 128)],
                                          gring.at[h, k], sems[h]).wait()

            def extract(c, h):
                lv = chunk_idx(c) & 127
                dbase = (c * CHUNK + lanes) * EMB
                for t in range(EMB):
                    vals = plsc.load_gather(
                        gring.at[h],
                        [lanes, jnp.full((16,), t, jnp.int32), lv])
                    df = dbase + t
                    plsc.store_scatter(
                        stage,
                        [lax.shift_right_logical(df, 7), df & 127],
                        vals)

            fire(0, 0)

            def pair(p, _):
                a = p * 2
                fire(a + 1, 1)
                drain(0)
                extract(a, 0)

                @pl.when(p < nch // 2 - 1)
                def _fire_next():
                    fire(a + 2, 0)

                drain(1)
                extract(a + 1, 1)
                return _

            lax.fori_loop(0, nch // 2, pair, None)
            pltpu.sync_copy(stage, out_hbm.at[pl.ds(obase, opw)])

        run_table(utT_hbm, ui_hbm, uo_hbm, uix)
        run_table(itT_hbm, ii_hbm, io_hbm, iix)

    return sc_gather


def _tc_mlp_body(u_ref, v_ref, wa_ref, wb_ref, wc_ref, b1_ref, w2_ref, b2_ref,
                 o_ref):
    u = u_ref[...]
    v = v_ref[...]
    e = u * v
    h = (jnp.dot(e, wa_ref[...], preferred_element_type=jnp.float32)
         + jnp.dot(u, wb_ref[...], preferred_element_type=jnp.float32)
         + jnp.dot(v, wc_ref[...], preferred_element_type=jnp.float32)
         + b1_ref[...])
    h = jnp.maximum(h, 0.0)
    y = jnp.dot(h, w2_ref[...], preferred_element_type=jnp.float32) + b2_ref[...]
    o_ref[...] = jax.nn.sigmoid(y)


def kernel(group_inputs, user_inputs, item_inputs, user_table, item_table,
           W1, b1, W2, b2):
    del group_inputs
    utT = user_table.T
    itT = item_table.T
    uidx = user_inputs.astype(jnp.int32).reshape(B // 128, 128)
    iidx = item_inputs.astype(jnp.int32).reshape(B // 128, 128)
    uo_p, io_p = _make_sc_gather(user_table.shape[0])(utT, itT, uidx, iidx)

    eye = jnp.eye(PACK, dtype=jnp.float32)
    wa = jnp.kron(eye, W1[0:EMB])
    wb = jnp.kron(eye, W1[EMB:2 * EMB])
    wc = jnp.kron(eye, W1[2 * EMB:3 * EMB])
    b1t = jnp.tile(b1, PACK).reshape(1, PACK * 8)
    w2p = jnp.kron(eye, W2)
    b2t = jnp.tile(b2, PACK).reshape(1, PACK)

    y = pl.pallas_call(
        _tc_mlp_body,
        out_shape=jax.ShapeDtypeStruct((B // PACK, PACK), jnp.float32),
    )(uo_p, io_p, wa, wb, wc, b1t, w2p, b2t)
    return y.reshape(B, 1)
